# baseline (speedup 1.0000x reference)
import dataclasses
import functools

import jax
import jax.numpy as jnp
from jax import lax
from jax.experimental import pallas as pl
from jax.experimental.pallas import tpu as pltpu
from jax.experimental.pallas import tpu_sc as plsc

N_NODES = 10000
NPAD = 10240
E = 320000
EPAD = 327680
PAD_ROW = 10232

NC = 2
NS = 16
CHUNK = 512
SUB = 128
N_SUB = CHUNK // SUB
ROWS_PER_TILE = NPAD // NS
EDGES_PER_TILE = EPAD // (NC * NS)
N_CHUNKS = EDGES_PER_TILE // CHUNK
HIST_PER_TILE = EPAD // (NC * NS)
HIST_CHUNKS = HIST_PER_TILE // CHUNK

_mesh = plsc.VectorSubcoreMesh(core_axis_name="c", subcore_axis_name="s")

_sc_params = pltpu.CompilerParams()
if "needs_layout_passes" in pltpu.CompilerParams.__dataclass_fields__:
    _sc_params = dataclasses.replace(_sc_params, needs_layout_passes=False)


HROWS = EPAD // 128 // (NC * NS)
HCHR = 16
HCH = HROWS // HCHR


@jax.jit
def _degree_partials(e3):
    @functools.partial(
        pl.kernel,
        out_type=jax.ShapeDtypeStruct((NC * NS, NPAD), jnp.float32),
        mesh=_mesh,
        compiler_params=_sc_params,
        scratch_types=[
            pltpu.VMEM((NPAD,), jnp.float32),
            pltpu.VMEM((HCHR, 128), jnp.int32),
        ],
    )
    def hist_kernel(e_hbm, out_hbm, hist_v, idx_v):
        c = lax.axis_index("c")
        s = lax.axis_index("s")
        wid = s * NC + c
        dst_hbm = e_hbm.at[1]

        zeros16 = jnp.zeros((16,), jnp.float32)

        @pl.loop(0, NPAD, step=16)
        def _(i):
            hist_v[pl.ds(i, 16)] = zeros16

        ones16 = jnp.ones((16,), jnp.float32)
        row_base = wid * HROWS

        @pl.loop(0, HCH)
        def _(ch):
            pltpu.sync_copy(dst_hbm.at[pl.ds(row_base + ch * HCHR, HCHR)], idx_v)

            @pl.loop(0, HCHR)
            def _(r):
                @pl.loop(0, 128, step=16)
                def _(k):
                    idx = idx_v.at[r][pl.ds(k, 16)]
                    plsc.addupdate_scatter(hist_v, [idx], ones16)

        pltpu.sync_copy(hist_v, out_hbm.at[wid])

    return hist_kernel(e3)


N_SUBS_PER_TILE = EDGES_PER_TILE // SUB
IDX_CHUNK = 8
OUTER = N_SUBS_PER_TILE // (2 * IDX_CHUNK)


@jax.jit
def _scatter_pass(y2, e3):
    @functools.partial(
        pl.kernel,
        out_type=jax.ShapeDtypeStruct((NC, NPAD, 128), jnp.float32),
        mesh=_mesh,
        compiler_params=_sc_params,
        scratch_types=[
            pltpu.VMEM_SHARED((NPAD, 128), jnp.float32),
            pltpu.VMEM((IDX_CHUNK, SUB), jnp.int32),
            pltpu.VMEM((IDX_CHUNK, SUB), jnp.int32),
            pltpu.VMEM((IDX_CHUNK, SUB), jnp.int32),
            pltpu.VMEM((IDX_CHUNK, SUB), jnp.int32),
            pltpu.VMEM((SUB, 128), jnp.float32),
            pltpu.VMEM((SUB, 128), jnp.float32),
            pltpu.SemaphoreType.DMA,
            pltpu.SemaphoreType.DMA,
            pltpu.SemaphoreType.DMA,
            pltpu.SemaphoreType.DMA,
            pltpu.SemaphoreType.DMA,
            pltpu.SemaphoreType.DMA,
            pltpu.SemaphoreType.DMA,
            pltpu.SemaphoreType.DMA,
        ],
    )
    def scatter_kernel(y_hbm, e_hbm, z_hbm, z_sp,
                       sidx_a, sidx_b, didx_a, didx_b, rows0, rows1,
                       gsem0, gsem1, ssem0, ssem1, ias, ibs, iad, ibd):
        c = lax.axis_index("c")
        s = lax.axis_index("s")

        r0 = s * ROWS_PER_TILE

        @pl.when(c == 0)
        def _():
            pltpu.sync_copy(
                y_hbm.at[pl.ds(r0, ROWS_PER_TILE)],
                z_sp.at[pl.ds(r0, ROWS_PER_TILE)],
            )

        @pl.when(c == 1)
        def _():
            zeros16 = jnp.zeros((16,), jnp.float32)

            @pl.loop(0, SUB)
            def _(i):
                @pl.loop(0, 128, step=16)
                def _(j):
                    rows0[i, pl.ds(j, 16)] = zeros16

            @pl.loop(0, ROWS_PER_TILE // SUB)
            def _(b):
                pltpu.sync_copy(rows0, z_sp.at[pl.ds(r0 + b * SUB, SUB)])

        plsc.subcore_barrier()

        y0 = y_hbm
        src_hbm = e_hbm.at[0]
        dst_hbm = e_hbm.at[1]
        tbase = (c * NS + s) * (EDGES_PER_TILE // 128)
        rows = (rows0, rows1)
        gsem = (gsem0, gsem1)
        ssem = (ssem0, ssem1)
        sidx = (sidx_a, sidx_b)
        didx = (didx_a, didx_b)

        def idx_chunk_refs(m):
            return (src_hbm.at[pl.ds(tbase + m * IDX_CHUNK, IDX_CHUNK)],
                    dst_hbm.at[pl.ds(tbase + m * IDX_CHUNK, IDX_CHUNK)])

        s_ref, d_ref = idx_chunk_refs(0)
        pltpu.sync_copy(s_ref, sidx_a)
        pltpu.sync_copy(d_ref, didx_a)
        s_ref, d_ref = idx_chunk_refs(1)
        pltpu.async_copy(s_ref, sidx_b, ibs)
        pltpu.async_copy(d_ref, didx_b, ibd)
        pltpu.async_copy(y0.at[sidx_a.at[0]], rows0, gsem0)
        pltpu.async_copy(y0.at[sidx_a.at[1]], rows1, gsem1)

        @pl.loop(0, OUTER)
        def _(q):
            not_last = q < OUTER - 1
            for k in range(2 * IDX_CHUNK):
                r = k % 2
                half = k // IDX_CHUNK
                row = k % IDX_CHUNK
                pltpu.make_async_copy(
                    y0.at[sidx[half].at[row]], rows[r], gsem[r]
                ).wait()
                if k == IDX_CHUNK - 1:
                    @pl.when(not_last)
                    def _():
                        s_ref, d_ref = idx_chunk_refs(2 * q + 2)
                        pltpu.async_copy(s_ref, sidx_a, ias)
                        pltpu.async_copy(d_ref, didx_a, iad)
                if k == 2 * IDX_CHUNK - 1:
                    @pl.when(not_last)
                    def _():
                        s_ref, d_ref = idx_chunk_refs(2 * q + 3)
                        pltpu.async_copy(s_ref, sidx_b, ibs)
                        pltpu.async_copy(d_ref, didx_b, ibd)
                pltpu.async_copy(
                    rows[r], z_sp.at[didx[half].at[row]], ssem[r], add=True
                ).wait()
                k2 = k + 2
                if k2 < 2 * IDX_CHUNK:
                    if k == IDX_CHUNK - 2:
                        s_ref, d_ref = idx_chunk_refs(0)
                        pltpu.make_async_copy(s_ref, sidx_b, ibs).wait()
                        pltpu.make_async_copy(d_ref, didx_b, ibd).wait()
                    h2 = k2 // IDX_CHUNK
                    pltpu.async_copy(
                        y0.at[sidx[h2].at[k2 % IDX_CHUNK]], rows[r], gsem[r]
                    )
                else:
                    @pl.when(not_last)
                    def _():
                        if k == 2 * IDX_CHUNK - 2:
                            s_ref, d_ref = idx_chunk_refs(0)
                            pltpu.make_async_copy(s_ref, sidx_a, ias).wait()
                            pltpu.make_async_copy(d_ref, didx_a, iad).wait()
                        pltpu.async_copy(
                            y0.at[sidx[0].at[k2 - 2 * IDX_CHUNK]], rows[r], gsem[r]
                        )

        plsc.subcore_barrier()
        pltpu.sync_copy(
            z_sp.at[pl.ds(r0, ROWS_PER_TILE)],
            z_hbm.at[c].at[pl.ds(r0, ROWS_PER_TILE)],
        )

    return scatter_kernel(y2, e3)


_BLK = 1024
_NBLK = NPAD // _BLK
_FBLK = 1000


@jax.jit
def _tc_scale(partials, x_pad):
    def body(p_ref, x_ref, y_ref, d_ref):
        deg = jnp.sum(p_ref[...], axis=0) + 1.0
        dinv = lax.rsqrt(deg)
        y_ref[...] = x_ref[...] * dinv[:, None]
        d_ref[...] = jnp.broadcast_to(dinv[:, None], (_BLK, 8))

    return pl.pallas_call(
        body,
        grid=(_NBLK,),
        in_specs=[
            pl.BlockSpec((NC * NS, _BLK), lambda i: (0, i)),
            pl.BlockSpec((_BLK, 128), lambda i: (i, 0)),
        ],
        out_specs=[
            pl.BlockSpec((_BLK, 128), lambda i: (i, 0)),
            pl.BlockSpec((_BLK, 8), lambda i: (i, 0)),
        ],
        out_shape=[
            jax.ShapeDtypeStruct((NPAD, 128), jnp.float32),
            jax.ShapeDtypeStruct((NPAD, 8), jnp.float32),
        ],
    )(partials, x_pad)


@jax.jit
def _tc_mid(z, dinv8, W1, b1, W2):
    def body(z_ref, d_ref, w1_ref, b1_ref, w2_ref, y2_ref):
        dinv = d_ref[:, :1]
        p = (z_ref[0] + z_ref[1]) * dinv
        h1 = jnp.maximum(
            jnp.dot(p, w1_ref[...], preferred_element_type=jnp.float32)
            + b1_ref[...],
            0.0,
        )
        q = jnp.dot(h1, w2_ref[...], preferred_element_type=jnp.float32)
        y2_ref[...] = q * dinv

    return pl.pallas_call(
        body,
        grid=(_NBLK,),
        in_specs=[
            pl.BlockSpec((NC, _BLK, 128), lambda i: (0, i, 0)),
            pl.BlockSpec((_BLK, 8), lambda i: (i, 0)),
            pl.BlockSpec((128, 256), lambda i: (0, 0)),
            pl.BlockSpec((1, 256), lambda i: (0, 0)),
            pl.BlockSpec((256, 128), lambda i: (0, 0)),
        ],
        out_specs=pl.BlockSpec((_BLK, 128), lambda i: (i, 0)),
        out_shape=jax.ShapeDtypeStruct((NPAD, 128), jnp.float32),
    )(z, dinv8, W1, b1, W2)


@jax.jit
def _tc_final(z2, dinv8, b2):
    def body(z_ref, d_ref, b2_ref, o_ref):
        dinv = d_ref[:, :1]
        h = (z_ref[0] + z_ref[1]) * dinv
        o_ref[...] = jnp.maximum(h + b2_ref[...], 0.0)

    return pl.pallas_call(
        body,
        grid=(N_NODES // _FBLK,),
        in_specs=[
            pl.BlockSpec((NC, _FBLK, 128), lambda i: (0, i, 0)),
            pl.BlockSpec((_FBLK, 8), lambda i: (i, 0)),
            pl.BlockSpec((1, 128), lambda i: (0, 0)),
        ],
        out_specs=pl.BlockSpec((_FBLK, 128), lambda i: (i, 0)),
        out_shape=jax.ShapeDtypeStruct((N_NODES, 128), jnp.float32),
    )(z2, dinv8, b2)


@jax.jit
def kernel(features, edges, W1, b1, W2, b2):
    x_pad = jnp.zeros((NPAD, 128), jnp.float32).at[:N_NODES].set(features)
    pad1 = N_NODES + (jnp.arange(EPAD - E, dtype=jnp.int32) % (NPAD - N_NODES))
    e_pad = jnp.concatenate([edges, jnp.stack([pad1, pad1])], axis=1)
    e3 = e_pad.reshape(2, EPAD // 128, 128)
    partials = _degree_partials(e3)
    y, dinv8 = _tc_scale(partials, x_pad)
    z1 = _scatter_pass(y, e3)
    y2 = _tc_mid(z1, dinv8, W1, b1.reshape(1, 256), W2)
    z2 = _scatter_pass(y2, e3)
    return _tc_final(z2, dinv8, b2.reshape(1, 128))


# speedup vs baseline: 1.0205x; 1.0205x over previous
import dataclasses
import functools

import jax
import jax.numpy as jnp
from jax import lax
from jax.experimental import pallas as pl
from jax.experimental.pallas import tpu as pltpu
from jax.experimental.pallas import tpu_sc as plsc

N_NODES = 10000
NPAD = 10240
E = 320000
EPAD = 327680
PAD_ROW = 10232

NC = 2
NS = 16
CHUNK = 512
SUB = 128
N_SUB = CHUNK // SUB
ROWS_PER_TILE = NPAD // NS
EDGES_PER_TILE = EPAD // (NC * NS)
N_CHUNKS = EDGES_PER_TILE // CHUNK
HIST_PER_TILE = EPAD // (NC * NS)
HIST_CHUNKS = HIST_PER_TILE // CHUNK

_mesh = plsc.VectorSubcoreMesh(core_axis_name="c", subcore_axis_name="s")

_sc_params = pltpu.CompilerParams()
if "needs_layout_passes" in pltpu.CompilerParams.__dataclass_fields__:
    _sc_params = dataclasses.replace(_sc_params, needs_layout_passes=False)


HROWS = EPAD // 128 // (NC * NS)
HCHR = 16
HCH = HROWS // HCHR


@jax.jit
def _degree_partials(e3):
    @functools.partial(
        pl.kernel,
        out_type=jax.ShapeDtypeStruct((NC * NS, NPAD), jnp.float32),
        mesh=_mesh,
        compiler_params=_sc_params,
        scratch_types=[
            pltpu.VMEM((NPAD,), jnp.float32),
            pltpu.VMEM((HCHR, 128), jnp.int32),
        ],
    )
    def hist_kernel(e_hbm, out_hbm, hist_v, idx_v):
        c = lax.axis_index("c")
        s = lax.axis_index("s")
        wid = s * NC + c
        dst_hbm = e_hbm.at[1]

        zeros16 = jnp.zeros((16,), jnp.float32)

        @pl.loop(0, NPAD, step=16)
        def _(i):
            hist_v[pl.ds(i, 16)] = zeros16

        ones16 = jnp.ones((16,), jnp.float32)
        row_base = wid * HROWS

        @pl.loop(0, HCH)
        def _(ch):
            pltpu.sync_copy(dst_hbm.at[pl.ds(row_base + ch * HCHR, HCHR)], idx_v)

            @pl.loop(0, HCHR)
            def _(r):
                @pl.loop(0, 128, step=16)
                def _(k):
                    idx = idx_v.at[r][pl.ds(k, 16)]
                    plsc.addupdate_scatter(hist_v, [idx], ones16)

        pltpu.sync_copy(hist_v, out_hbm.at[wid])

    return hist_kernel(e3)


SSUB = 64
NSLOT = 4
N_SUBS_PER_TILE = EDGES_PER_TILE // SSUB
IDX_ROWS = 8
IDX_CHUNK = 2 * IDX_ROWS
KSTEPS = 2 * IDX_CHUNK
OUTER = N_SUBS_PER_TILE // KSTEPS


@jax.jit
def _scatter_pass(y2, e3):
    @functools.partial(
        pl.kernel,
        out_type=jax.ShapeDtypeStruct((NC, NPAD, 128), jnp.float32),
        mesh=_mesh,
        compiler_params=_sc_params,
        scratch_types=[
            pltpu.VMEM_SHARED((NPAD, 128), jnp.float32),
            pltpu.VMEM((IDX_ROWS, 128), jnp.int32),
            pltpu.VMEM((IDX_ROWS, 128), jnp.int32),
            pltpu.VMEM((IDX_ROWS, 128), jnp.int32),
            pltpu.VMEM((IDX_ROWS, 128), jnp.int32),
            [pltpu.VMEM((SSUB, 128), jnp.float32) for _ in range(NSLOT)],
            [pltpu.VMEM((SSUB,), jnp.int32) for _ in range(NSLOT)],
            [pltpu.VMEM((SSUB,), jnp.int32) for _ in range(NSLOT)],
            [pltpu.SemaphoreType.DMA for _ in range(NSLOT)],
            [pltpu.SemaphoreType.DMA for _ in range(NSLOT)],
            pltpu.SemaphoreType.DMA,
            pltpu.SemaphoreType.DMA,
            pltpu.SemaphoreType.DMA,
            pltpu.SemaphoreType.DMA,
        ],
    )
    def scatter_kernel(y_hbm, e_hbm, z_hbm, z_sp,
                       sidx_a, sidx_b, didx_a, didx_b,
                       rows, stg_s, stg_d, gsem, ssem, ias, ibs, iad, ibd):
        c = lax.axis_index("c")
        s = lax.axis_index("s")

        r0 = s * ROWS_PER_TILE

        @pl.when(c == 0)
        def _():
            pltpu.sync_copy(
                y_hbm.at[pl.ds(r0, ROWS_PER_TILE)],
                z_sp.at[pl.ds(r0, ROWS_PER_TILE)],
            )

        @pl.when(c == 1)
        def _():
            zeros16 = jnp.zeros((16,), jnp.float32)

            @pl.loop(0, SSUB)
            def _(i):
                @pl.loop(0, 128, step=16)
                def _(j):
                    rows[0][i, pl.ds(j, 16)] = zeros16

            @pl.loop(0, ROWS_PER_TILE // SSUB)
            def _(b):
                pltpu.sync_copy(rows[0], z_sp.at[pl.ds(r0 + b * SSUB, SSUB)])

        plsc.subcore_barrier()

        src_hbm = e_hbm.at[0]
        dst_hbm = e_hbm.at[1]
        tbase = (c * NS + s) * (EDGES_PER_TILE // 128)
        sidx = (sidx_a, sidx_b)
        didx = (didx_a, didx_b)

        def idx_chunk_refs(m):
            return (src_hbm.at[pl.ds(tbase + m * IDX_ROWS, IDX_ROWS)],
                    dst_hbm.at[pl.ds(tbase + m * IDX_ROWS, IDX_ROWS)])

        def stage(idx_slots, stg, pos, r):
            half, rowk = pos // IDX_CHUNK, pos % IDX_CHUNK
            row, col = rowk // 2, (rowk % 2) * SSUB
            for j in range(0, SSUB, 16):
                stg[r][pl.ds(j, 16)] = idx_slots[half].at[row][pl.ds(col + j, 16)]

        def gather_desc(r):
            return pltpu.make_async_copy(y_hbm.at[stg_s[r]], rows[r], gsem[r])

        def issue_gather(pos, r):
            stage(sidx, stg_s, pos, r)
            pltpu.async_copy(y_hbm.at[stg_s[r]], rows[r], gsem[r])

        def issue_scatter(pos, r):
            stage(didx, stg_d, pos, r)
            pltpu.async_copy(rows[r], z_sp.at[stg_d[r]], ssem[r], add=True)

        def scatter_wait(r):
            pltpu.make_async_copy(rows[r], z_sp.at[stg_d[r]], ssem[r]).wait()

        s_ref, d_ref = idx_chunk_refs(0)
        pltpu.sync_copy(s_ref, sidx_a)
        pltpu.sync_copy(d_ref, didx_a)
        s_ref, d_ref = idx_chunk_refs(1)
        pltpu.async_copy(s_ref, sidx_b, ibs)
        pltpu.async_copy(d_ref, didx_b, ibd)

        @pl.loop(0, OUTER)
        def _(q):
            not_first = q > 0
            not_last = q < OUTER - 1
            for k in range(KSTEPS):
                r = k % NSLOT
                if k < NSLOT:
                    @pl.when(not_first)
                    def _():
                        scatter_wait(r)
                else:
                    scatter_wait(r)
                if k == 0:
                    @pl.when(not_first)
                    def _():
                        s_ref, d_ref = idx_chunk_refs(0)
                        pltpu.make_async_copy(s_ref, sidx_a, ias).wait()
                        pltpu.make_async_copy(d_ref, didx_a, iad).wait()
                if k == IDX_CHUNK:
                    s_ref, d_ref = idx_chunk_refs(0)
                    pltpu.make_async_copy(s_ref, sidx_b, ibs).wait()
                    pltpu.make_async_copy(d_ref, didx_b, ibd).wait()
                issue_gather(k, r)
                if k == 3:
                    @pl.when(not_first)
                    def _():
                        s_ref, d_ref = idx_chunk_refs(2 * q + 1)
                        pltpu.async_copy(s_ref, sidx_b, ibs)
                        pltpu.async_copy(d_ref, didx_b, ibd)
                if k == 19:
                    @pl.when(not_last)
                    def _():
                        s_ref, d_ref = idx_chunk_refs(2 * q + 2)
                        pltpu.async_copy(s_ref, sidx_a, ias)
                        pltpu.async_copy(d_ref, didx_a, iad)
                k2 = k - 2
                if k2 >= 0:
                    r2 = k2 % NSLOT
                    gather_desc(r2).wait()
                    issue_scatter(k2, r2)
                else:
                    @pl.when(not_first)
                    def _():
                        k2p = k2 + KSTEPS
                        r2 = k2p % NSLOT
                        gather_desc(r2).wait()
                        issue_scatter(k2p, r2)

        for k2 in (KSTEPS - 2, KSTEPS - 1):
            r2 = k2 % NSLOT
            gather_desc(r2).wait()
            issue_scatter(k2, r2)
        for r in range(NSLOT):
            scatter_wait(r)

        plsc.subcore_barrier()
        pltpu.sync_copy(
            z_sp.at[pl.ds(r0, ROWS_PER_TILE)],
            z_hbm.at[c].at[pl.ds(r0, ROWS_PER_TILE)],
        )

    return scatter_kernel(y2, e3)


_BLK = 1024
_NBLK = NPAD // _BLK
_FBLK = 1000


@jax.jit
def _tc_scale(partials, x_pad):
    def body(p_ref, x_ref, y_ref, d_ref):
        deg = jnp.sum(p_ref[...], axis=0) + 1.0
        dinv = lax.rsqrt(deg)
        y_ref[...] = x_ref[...] * dinv[:, None]
        d_ref[...] = jnp.broadcast_to(dinv[:, None], (_BLK, 8))

    return pl.pallas_call(
        body,
        grid=(_NBLK,),
        in_specs=[
            pl.BlockSpec((NC * NS, _BLK), lambda i: (0, i)),
            pl.BlockSpec((_BLK, 128), lambda i: (i, 0)),
        ],
        out_specs=[
            pl.BlockSpec((_BLK, 128), lambda i: (i, 0)),
            pl.BlockSpec((_BLK, 8), lambda i: (i, 0)),
        ],
        out_shape=[
            jax.ShapeDtypeStruct((NPAD, 128), jnp.float32),
            jax.ShapeDtypeStruct((NPAD, 8), jnp.float32),
        ],
    )(partials, x_pad)


@jax.jit
def _tc_mid(z, dinv8, W1, b1, W2):
    def body(z_ref, d_ref, w1_ref, b1_ref, w2_ref, y2_ref):
        dinv = d_ref[:, :1]
        p = (z_ref[0] + z_ref[1]) * dinv
        h1 = jnp.maximum(
            jnp.dot(p, w1_ref[...], preferred_element_type=jnp.float32)
            + b1_ref[...],
            0.0,
        )
        q = jnp.dot(h1, w2_ref[...], preferred_element_type=jnp.float32)
        y2_ref[...] = q * dinv

    return pl.pallas_call(
        body,
        grid=(_NBLK,),
        in_specs=[
            pl.BlockSpec((NC, _BLK, 128), lambda i: (0, i, 0)),
            pl.BlockSpec((_BLK, 8), lambda i: (i, 0)),
            pl.BlockSpec((128, 256), lambda i: (0, 0)),
            pl.BlockSpec((1, 256), lambda i: (0, 0)),
            pl.BlockSpec((256, 128), lambda i: (0, 0)),
        ],
        out_specs=pl.BlockSpec((_BLK, 128), lambda i: (i, 0)),
        out_shape=jax.ShapeDtypeStruct((NPAD, 128), jnp.float32),
    )(z, dinv8, W1, b1, W2)


@jax.jit
def _tc_final(z2, dinv8, b2):
    def body(z_ref, d_ref, b2_ref, o_ref):
        dinv = d_ref[:, :1]
        h = (z_ref[0] + z_ref[1]) * dinv
        o_ref[...] = jnp.maximum(h + b2_ref[...], 0.0)

    return pl.pallas_call(
        body,
        grid=(N_NODES // _FBLK,),
        in_specs=[
            pl.BlockSpec((NC, _FBLK, 128), lambda i: (0, i, 0)),
            pl.BlockSpec((_FBLK, 8), lambda i: (i, 0)),
            pl.BlockSpec((1, 128), lambda i: (0, 0)),
        ],
        out_specs=pl.BlockSpec((_FBLK, 128), lambda i: (i, 0)),
        out_shape=jax.ShapeDtypeStruct((N_NODES, 128), jnp.float32),
    )(z2, dinv8, b2)


@jax.jit
def kernel(features, edges, W1, b1, W2, b2):
    x_pad = jnp.zeros((NPAD, 128), jnp.float32).at[:N_NODES].set(features)
    pad1 = N_NODES + (jnp.arange(EPAD - E, dtype=jnp.int32) % (NPAD - N_NODES))
    e_pad = jnp.concatenate([edges, jnp.stack([pad1, pad1])], axis=1)
    e3 = e_pad.reshape(2, EPAD // 128, 128)
    partials = _degree_partials(e3)
    y, dinv8 = _tc_scale(partials, x_pad)
    z1 = _scatter_pass(y, e3)
    y2 = _tc_mid(z1, dinv8, W1, b1.reshape(1, 256), W2)
    z2 = _scatter_pass(y2, e3)
    return _tc_final(z2, dinv8, b2.reshape(1, 128))
